# baseline (speedup 1.0000x reference)
import dataclasses
import functools

import jax
import jax.numpy as jnp
from jax import lax
from jax.experimental import pallas as pl
from jax.experimental.pallas import tpu as pltpu
from jax.experimental.pallas import tpu_sc as plsc

D = 1024
_NC = 2
_NS = 16
_NW = _NC * _NS
_SLAB = D // _NW
_L = 16
_TM = 512
_PREC_W = lax.Precision.DEFAULT
_PREC_H = lax.Precision.DEFAULT


def _build_s_body(rows_hbm, cols_hbm, vals_hbm, out_hbm, r_v, c_v, v_v, acc_v):
    nnz_pad = r_v.shape[0]
    wid = lax.axis_index("s") * _NC + lax.axis_index("c")
    base = wid * _SLAB
    pltpu.sync_copy(rows_hbm, r_v)
    pltpu.sync_copy(cols_hbm, c_v)
    pltpu.sync_copy(vals_hbm, v_v)

    zeros = jnp.zeros((_L,), jnp.float32)

    @pl.loop(0, _SLAB * D, step=_L)
    def _zero(j):
        acc_v.at[pl.ds(j, _L)][...] = zeros

    @pl.loop(0, nnz_pad, step=_L)
    def _scatter(t):
        sl = pl.ds(t, _L)
        r = r_v[sl]
        c = c_v[sl]
        v = v_v[sl]
        lr = r - base
        m1 = (lr >= 0) & (lr < _SLAB)
        idx1 = jnp.where(m1, lr * D + c, 0)
        plsc.store_scatter(acc_v, [idx1], v, mask=m1)
        lc = c - base
        m2 = (lc >= 0) & (lc < _SLAB)
        idx2 = jnp.where(m2, lc * D + r, 0)
        plsc.store_scatter(acc_v, [idx2], -v, mask=m2)

    pltpu.sync_copy(acc_v, out_hbm.at[pl.ds(base * D, _SLAB * D)])


@functools.lru_cache(maxsize=None)
def _build_s_kernel(nnz_pad):
    cp = pltpu.CompilerParams()
    if "needs_layout_passes" in pltpu.CompilerParams.__dataclass_fields__:
        cp = dataclasses.replace(cp, needs_layout_passes=False)
    return pl.kernel(
        _build_s_body,
        out_type=jax.ShapeDtypeStruct((D * D,), jnp.float32),
        mesh=plsc.VectorSubcoreMesh(core_axis_name="c", subcore_axis_name="s"),
        compiler_params=cp,
        scratch_types=[
            pltpu.VMEM((nnz_pad,), jnp.int32),
            pltpu.VMEM((nnz_pad,), jnp.int32),
            pltpu.VMEM((nnz_pad,), jnp.float32),
            pltpu.VMEM((_SLAB * D,), jnp.float32),
        ],
    )


def _cayley_w_body(s_ref, w2_ref):
    S = s_ref[...]
    S2 = jnp.dot(S, S, preferred_element_type=jnp.float32, precision=_PREC_W)
    T = S + S2
    T = T + jnp.dot(S2, T, preferred_element_type=jnp.float32, precision=_PREC_W)
    S4 = jnp.dot(S2, S2, preferred_element_type=jnp.float32, precision=_PREC_W)
    T = T + jnp.dot(S4, T, preferred_element_type=jnp.float32, precision=_PREC_W)
    rr = lax.broadcasted_iota(jnp.int32, (D, D), 0)
    cc = lax.broadcasted_iota(jnp.int32, (D, D), 1)
    W = jnp.where(rr == cc, 1.0, 0.0) + 2.0 * T
    w2_ref[:D] = W
    Wp = jnp.roll(W, 1, axis=0)
    Wm = jnp.roll(W, -1, axis=0)
    even_row = (rr & 1) == 0
    w2_ref[D:] = jnp.where(even_row, Wm, Wp)


def _rope_mm_body(ang_ref, q_ref, k_ref, w2_ref, oq_ref, ok_ref):
    ang = ang_ref[...]
    cosv = jnp.cos(ang)
    sinv = jnp.sin(ang)
    W = w2_ref[:D]
    Wsw = w2_ref[D:]

    def rope_mm(h):
        a = jnp.dot(h * cosv, W, preferred_element_type=jnp.float32,
                    precision=_PREC_H)
        b = jnp.dot(h * sinv, Wsw, preferred_element_type=jnp.float32,
                    precision=_PREC_H)
        return a + b

    oq_ref[0] = rope_mm(q_ref[0])
    ok_ref[0] = rope_mm(k_ref[0])


def _rope_mm_call(B, N):
    grid = (B, N // _TM)
    return pl.pallas_call(
        _rope_mm_body,
        grid=grid,
        in_specs=[
            pl.BlockSpec((_TM, D), lambda b, n: (n, 0)),
            pl.BlockSpec((1, _TM, D), lambda b, n: (b, n, 0)),
            pl.BlockSpec((1, _TM, D), lambda b, n: (b, n, 0)),
            pl.BlockSpec((2 * D, D), lambda b, n: (0, 0)),
        ],
        out_specs=[
            pl.BlockSpec((1, _TM, D), lambda b, n: (b, n, 0)),
            pl.BlockSpec((1, _TM, D), lambda b, n: (b, n, 0)),
        ],
        out_shape=[
            jax.ShapeDtypeStruct((B, N, D), jnp.float32),
            jax.ShapeDtypeStruct((B, N, D), jnp.float32),
        ],
        compiler_params=pltpu.CompilerParams(
            dimension_semantics=("parallel", "arbitrary"),
        ),
    )


def kernel(q, k, pos, s_params, freqs, rows, cols):
    B, N, d = q.shape
    nnz = rows.shape[0]
    pad = (-nnz) % _L
    rows_p = jnp.concatenate(
        [rows.astype(jnp.int32), jnp.zeros((pad,), jnp.int32)])
    cols_p = jnp.concatenate(
        [cols.astype(jnp.int32), jnp.zeros((pad,), jnp.int32)])
    vals_p = jnp.concatenate(
        [s_params.astype(jnp.float32), jnp.zeros((pad,), jnp.float32)])

    s_flat = _build_s_kernel(nnz + pad)(rows_p, cols_p, vals_p)
    S = s_flat.reshape(D, D)

    W2 = pl.pallas_call(
        _cayley_w_body,
        out_shape=jax.ShapeDtypeStruct((2 * D, D), jnp.float32),
    )(S)

    alt = jnp.where(jnp.arange(D) % 2 == 0, 1.0, -1.0).astype(jnp.float32)
    ff2 = (jnp.repeat(freqs, 2) * alt).reshape(1, D)
    ang = pos.reshape(N, 1) * ff2
    q_out, k_out = _rope_mm_call(B, N)(ang, q, k, W2)
    return (q_out, k_out)


# speedup vs baseline: 1.0068x; 1.0068x over previous
import dataclasses
import functools

import jax
import jax.numpy as jnp
from jax import lax
from jax.experimental import pallas as pl
from jax.experimental.pallas import tpu as pltpu
from jax.experimental.pallas import tpu_sc as plsc

D = 1024
_NC = 2
_NS = 16
_NW = _NC * _NS
_SLAB = D // _NW
_L = 16
_TM = 512
_PREC_W = lax.Precision.DEFAULT
_PREC_H = lax.Precision.DEFAULT


def _build_s_body(rows_hbm, cols_hbm, vals_hbm, out_hbm, r_v, c_v, v_v, acc_v):
    nnz_pad = r_v.shape[0]
    wid = lax.axis_index("s") * _NC + lax.axis_index("c")
    base = wid * _SLAB
    pltpu.sync_copy(rows_hbm, r_v)
    pltpu.sync_copy(cols_hbm, c_v)
    pltpu.sync_copy(vals_hbm, v_v)

    zeros = jnp.zeros((_L,), jnp.float32)

    @pl.loop(0, _SLAB * D, step=_L)
    def _zero(j):
        acc_v.at[pl.ds(j, _L)][...] = zeros

    @pl.loop(0, nnz_pad, step=_L)
    def _scatter(t):
        sl = pl.ds(t, _L)
        r = r_v[sl]
        c = c_v[sl]
        v = v_v[sl]
        lr = r - base
        m1 = (lr >= 0) & (lr < _SLAB)
        idx1 = jnp.where(m1, lr * D + c, 0)
        plsc.store_scatter(acc_v, [idx1], v, mask=m1)
        lc = c - base
        m2 = (lc >= 0) & (lc < _SLAB)
        idx2 = jnp.where(m2, lc * D + r, 0)
        plsc.store_scatter(acc_v, [idx2], -v, mask=m2)

    pltpu.sync_copy(acc_v, out_hbm.at[pl.ds(base * D, _SLAB * D)])


@functools.lru_cache(maxsize=None)
def _build_s_kernel(nnz_pad):
    cp = pltpu.CompilerParams()
    if "needs_layout_passes" in pltpu.CompilerParams.__dataclass_fields__:
        cp = dataclasses.replace(cp, needs_layout_passes=False)
    return pl.kernel(
        _build_s_body,
        out_type=jax.ShapeDtypeStruct((D * D,), jnp.float32),
        mesh=plsc.VectorSubcoreMesh(core_axis_name="c", subcore_axis_name="s"),
        compiler_params=cp,
        scratch_types=[
            pltpu.VMEM((nnz_pad,), jnp.int32),
            pltpu.VMEM((nnz_pad,), jnp.int32),
            pltpu.VMEM((nnz_pad,), jnp.float32),
            pltpu.VMEM((_SLAB * D,), jnp.float32),
        ],
    )


def _cayley_w_body(s_ref, w2_ref):
    S = s_ref[...]
    S2 = jnp.dot(S, S, preferred_element_type=jnp.float32, precision=_PREC_W)
    T = S + S2
    T = T + jnp.dot(S2, T, preferred_element_type=jnp.float32, precision=_PREC_W)
    S4 = jnp.dot(S2, S2, preferred_element_type=jnp.float32, precision=_PREC_W)
    T = T + jnp.dot(S4, T, preferred_element_type=jnp.float32, precision=_PREC_W)
    rr = lax.broadcasted_iota(jnp.int32, (D, D), 0)
    cc = lax.broadcasted_iota(jnp.int32, (D, D), 1)
    W = jnp.where(rr == cc, 1.0, 0.0) + 2.0 * T
    w2_ref[:D] = W
    Wp = jnp.roll(W, 1, axis=0)
    Wm = jnp.roll(W, -1, axis=0)
    even_row = (rr & 1) == 0
    w2_ref[D:] = jnp.where(even_row, Wm, Wp)


def _rope_mm_body(pos_ref, ff_ref, q_ref, k_ref, w2_ref, oq_ref, ok_ref):
    ang = jnp.dot(pos_ref[...], ff_ref[...],
                  preferred_element_type=jnp.float32, precision=_PREC_H)
    cosv = jnp.cos(ang)
    sinv = jnp.sin(ang)
    W = w2_ref[:D]
    Wsw = w2_ref[D:]

    def rope_mm(h):
        a = jnp.dot(h * cosv, W, preferred_element_type=jnp.float32,
                    precision=_PREC_H)
        b = jnp.dot(h * sinv, Wsw, preferred_element_type=jnp.float32,
                    precision=_PREC_H)
        return a + b

    oq_ref[0] = rope_mm(q_ref[0])
    ok_ref[0] = rope_mm(k_ref[0])


def _rope_mm_call(B, N):
    grid = (B, N // _TM)
    return pl.pallas_call(
        _rope_mm_body,
        grid=grid,
        in_specs=[
            pl.BlockSpec((_TM, 16), lambda b, n: (n, 0)),
            pl.BlockSpec((16, D), lambda b, n: (0, 0)),
            pl.BlockSpec((1, _TM, D), lambda b, n: (b, n, 0)),
            pl.BlockSpec((1, _TM, D), lambda b, n: (b, n, 0)),
            pl.BlockSpec((2 * D, D), lambda b, n: (0, 0)),
        ],
        out_specs=[
            pl.BlockSpec((1, _TM, D), lambda b, n: (b, n, 0)),
            pl.BlockSpec((1, _TM, D), lambda b, n: (b, n, 0)),
        ],
        out_shape=[
            jax.ShapeDtypeStruct((B, N, D), jnp.float32),
            jax.ShapeDtypeStruct((B, N, D), jnp.float32),
        ],
        compiler_params=pltpu.CompilerParams(
            dimension_semantics=("parallel", "arbitrary"),
        ),
    )


def kernel(q, k, pos, s_params, freqs, rows, cols):
    B, N, d = q.shape
    nnz = rows.shape[0]
    pad = (-nnz) % _L
    rows_p = jnp.concatenate(
        [rows.astype(jnp.int32), jnp.zeros((pad,), jnp.int32)])
    cols_p = jnp.concatenate(
        [cols.astype(jnp.int32), jnp.zeros((pad,), jnp.int32)])
    vals_p = jnp.concatenate(
        [s_params.astype(jnp.float32), jnp.zeros((pad,), jnp.float32)])

    s_flat = _build_s_kernel(nnz + pad)(rows_p, cols_p, vals_p)
    S = s_flat.reshape(D, D)

    W2 = pl.pallas_call(
        _cayley_w_body,
        out_shape=jax.ShapeDtypeStruct((2 * D, D), jnp.float32),
    )(S)

    alt = jnp.where(jnp.arange(D) % 2 == 0, 1.0, -1.0).astype(jnp.float32)
    ff2 = (jnp.repeat(freqs, 2) * alt).reshape(D)

    def _split3(x):
        h1 = x.astype(jnp.bfloat16).astype(jnp.float32)
        r = x - h1
        h2 = r.astype(jnp.bfloat16).astype(jnp.float32)
        return h1, h2, r - h2

    p1, p2, p3 = _split3(pos)
    f1, f2, f3 = _split3(ff2)
    zcol = jnp.zeros((N,), jnp.float32)
    pos16 = jnp.stack(
        [p1, p1, p1, p2, p2, p2, p3, p3, p3,
         zcol, zcol, zcol, zcol, zcol, zcol, zcol], axis=1)
    zrow = jnp.zeros((D,), jnp.float32)
    ff16 = jnp.stack(
        [f1, f2, f3, f1, f2, f3, f1, f2, f3,
         zrow, zrow, zrow, zrow, zrow, zrow, zrow], axis=0)
    q_out, k_out = _rope_mm_call(B, N)(pos16, ff16, q, k, W2)
    return (q_out, k_out)
